# baseline (speedup 1.0000x reference)
import functools

import jax
import jax.numpy as jnp
from jax import lax
from jax.experimental import pallas as pl
from jax.experimental.pallas import tpu as pltpu
from jax.experimental.pallas import tpu_sc as plsc

NUM_NEWS = 1000000
NUM_CATEGORIES = 1000
NEWS_DIM = 64
CATEGORY_DIM = 16
BATCH = 4096
SEQ_LEN = 50
TOTAL = BATCH * SEQ_LEN
JOINT_DIM = NEWS_DIM + CATEGORY_DIM
ROW_PAD = 128
SEQ_PAD = 64

HALF_NEWS = NUM_NEWS // 2
NUM_CORES = 2
NUM_SUBCORES = 16
NW = NUM_CORES * NUM_SUBCORES
BATCH_W = BATCH // NW
LANES = 16
NBUF = 4
LEAD = 3


def _sc_body(nlo2_hbm, nhi2_hbm, cidx_hbm,
             newslo_hbm, newshi_hbm, cat_hbm, out_hbm,
             nlo2_v, nhi2_v, cidx_v, cat_v,
             pa0_v, pa1_v, pa2_v, pa3_v,
             pb0_v, pb1_v, pb2_v, pb3_v,
             stage0_v, stage1_v, stage2_v, stage3_v,
             gsem0, gsem1, gsem2, gsem3, wsem0, wsem1, wsem2, wsem3):
    cid = lax.axis_index("c")
    sid = lax.axis_index("s")
    wid = sid * NUM_CORES + cid
    base = wid * BATCH_W * SEQ_PAD
    pltpu.sync_copy(nlo2_hbm.at[pl.ds(base, BATCH_W * SEQ_PAD)], nlo2_v)
    pltpu.sync_copy(nhi2_hbm.at[pl.ds(base, BATCH_W * SEQ_PAD)], nhi2_v)
    pltpu.sync_copy(cidx_hbm.at[pl.ds(base, BATCH_W * SEQ_PAD)], cidx_v)
    pltpu.sync_copy(cat_hbm, cat_v)

    pas = (pa0_v, pa1_v, pa2_v, pa3_v)
    pbs = (pb0_v, pb1_v, pb2_v, pb3_v)
    stages = (stage0_v, stage1_v, stage2_v, stage3_v)
    gsems = (gsem0, gsem1, gsem2, gsem3)
    wsems = (wsem0, wsem1, wsem2, wsem3)

    def gather_copies(b, s):
        idx_lo = nlo2_v.at[pl.ds(b * SEQ_PAD, SEQ_LEN)]
        idx_hi = nhi2_v.at[pl.ds(b * SEQ_PAD, SEQ_LEN)]
        return (
            pltpu.make_async_copy(newslo_hbm.at[idx_lo], pas[s], gsems[s]),
            pltpu.make_async_copy(newshi_hbm.at[idx_hi], pbs[s], gsems[s]),
        )

    def write_copy(b, s):
        return pltpu.make_async_copy(stages[s],
                                     out_hbm.at[pl.ds(wid * BATCH_W + b, 1)],
                                     wsems[s])

    def merge(b, s):
        pva = pas[s]
        pvb = pbs[s]
        sv = stages[s]
        iota = lax.iota(jnp.int32, LANES)

        def row_body(r, carry):
            rsplat = jnp.full((LANES,), b * SEQ_PAD + r, jnp.int32)
            cp = plsc.load_gather(cidx_v, [rsplat])
            odd = (cp & (1 << 12)) != 0
            inhi = (cp & (1 << 13)) != 0
            cidv = cp & 0xFFF
            for k in range(NEWS_DIM // LANES):
                alo = pva[r, pl.ds(k * LANES, LANES)]
                ahi = pva[r, pl.ds(NEWS_DIM + k * LANES, LANES)]
                blo = pvb[r, pl.ds(k * LANES, LANES)]
                bhi = pvb[r, pl.ds(NEWS_DIM + k * LANES, LANES)]
                va = jnp.where(odd, ahi, alo)
                vb = jnp.where(odd, bhi, blo)
                sv[0, r, pl.ds(k * LANES, LANES)] = jnp.where(inhi, vb, va)
            cvals = plsc.load_gather(cat_v, [cidv * CATEGORY_DIM + iota])
            sv[0, r, pl.ds(NEWS_DIM, LANES)] = cvals
            return carry

        lax.fori_loop(0, SEQ_LEN, row_body, 0)

    for p in range(LEAD):
        for cp in gather_copies(p, p):
            cp.start()

    def pair_body(g, carry):
        for s in range(NBUF):
            b = g * NBUF + s
            nb = b + LEAD
            @pl.when(nb < BATCH_W)
            def _():
                for cp in gather_copies(nb, (s + LEAD) % NBUF):
                    cp.start()
            for cp in gather_copies(b, s):
                cp.wait()
            @pl.when(b >= NBUF)
            def _():
                write_copy(b - NBUF, s).wait()
            merge(b, s)
            write_copy(b, s).start()
        return carry

    lax.fori_loop(0, BATCH_W // NBUF, pair_body, 0)
    for p in range(NBUF):
        b = BATCH_W - NBUF + p
        write_copy(b, b % NBUF).wait()


@jax.jit
def _joint_embed(nlo2, nhi2, cat_idx, newslo, newshi, cat_flat):
    mesh = plsc.VectorSubcoreMesh(core_axis_name="c", subcore_axis_name="s")
    f = functools.partial(
        pl.kernel,
        mesh=mesh,
        out_type=jax.ShapeDtypeStruct((BATCH, SEQ_LEN, JOINT_DIM),
                                      jnp.float32),
        scratch_types=(
            [pltpu.VMEM((BATCH_W * SEQ_PAD,), jnp.int32)] * 3
            + [pltpu.VMEM((NUM_CATEGORIES * CATEGORY_DIM,), jnp.float32)]
            + [pltpu.VMEM((SEQ_LEN, ROW_PAD), jnp.float32)] * 8
            + [pltpu.VMEM((1, SEQ_LEN, JOINT_DIM), jnp.float32)] * 4
            + [pltpu.SemaphoreType.DMA] * 8
        ),
        compiler_params=pltpu.CompilerParams(needs_layout_passes=False),
    )(_sc_body)
    return f(nlo2, nhi2, cat_idx, newslo, newshi, cat_flat)


def kernel(news_ids, category_ids, news_table, category_table):
    pad = ((0, 0), (0, SEQ_PAD - SEQ_LEN))
    news_idx = jnp.pad(news_ids, pad).reshape(BATCH * SEQ_PAD)
    nlo2 = jnp.minimum(news_idx, HALF_NEWS - 1) >> 1
    nhi2 = (jnp.maximum(news_idx, HALF_NEWS) - HALF_NEWS) >> 1
    cat_idx = (jnp.pad(category_ids, pad)
               | ((news_idx & 1) << 12).reshape(BATCH, SEQ_PAD)
               | ((news_idx >= HALF_NEWS).astype(jnp.int32) << 13).reshape(
                   BATCH, SEQ_PAD)).reshape(BATCH * SEQ_PAD)
    newslo = news_table[:HALF_NEWS].reshape(HALF_NEWS // 2, ROW_PAD)
    newshi = news_table[HALF_NEWS:].reshape(HALF_NEWS // 2, ROW_PAD)
    cat_flat = category_table.reshape(NUM_CATEGORIES * CATEGORY_DIM)
    return _joint_embed(nlo2, nhi2, cat_idx, newslo, newshi, cat_flat)


# speedup vs baseline: 6.5456x; 6.5456x over previous
import functools

import jax
import jax.numpy as jnp
from jax import lax
from jax.experimental import pallas as pl
from jax.experimental.pallas import tpu as pltpu
from jax.experimental.pallas import tpu_sc as plsc

NUM_NEWS = 1000000
NUM_CATEGORIES = 1000
NEWS_DIM = 64
CATEGORY_DIM = 16
BATCH = 4096
SEQ_LEN = 50
TOTAL = BATCH * SEQ_LEN
JOINT_DIM = NEWS_DIM + CATEGORY_DIM
ROW_PAD = 128
SEQ_PAD = 64

NUM_CORES = 2
NUM_SUBCORES = 16
NW = NUM_CORES * NUM_SUBCORES
BATCH_W = BATCH // NW
LANES = 16
NBUF = 4
LEAD = 3


def _sc_body(nidx2_hbm, nidx_hbm, cidx_hbm, news_hbm, cat_hbm, out_hbm,
             nidx2_v, nidx_v, cidx_v, cat_v,
             pair0_v, pair1_v, pair2_v, pair3_v,
             stage0_v, stage1_v, stage2_v, stage3_v,
             gsem0, gsem1, gsem2, gsem3, wsem0, wsem1, wsem2, wsem3):
    cid = lax.axis_index("c")
    sid = lax.axis_index("s")
    wid = sid * NUM_CORES + cid
    base = wid * BATCH_W * SEQ_PAD
    pltpu.sync_copy(nidx2_hbm.at[pl.ds(base, BATCH_W * SEQ_PAD)], nidx2_v)
    pltpu.sync_copy(nidx_hbm.at[pl.ds(base, BATCH_W * SEQ_PAD)], nidx_v)
    pltpu.sync_copy(cidx_hbm.at[pl.ds(base, BATCH_W * SEQ_PAD)], cidx_v)
    pltpu.sync_copy(cat_hbm, cat_v)

    pairs = (pair0_v, pair1_v, pair2_v, pair3_v)
    stages = (stage0_v, stage1_v, stage2_v, stage3_v)
    gsems = (gsem0, gsem1, gsem2, gsem3)
    wsems = (wsem0, wsem1, wsem2, wsem3)

    def gather_copy(b, s):
        idx_n = nidx2_v.at[pl.ds(b * SEQ_PAD, SEQ_LEN)]
        return pltpu.make_async_copy(news_hbm.at[idx_n], pairs[s], gsems[s])

    def write_copy(b, s):
        return pltpu.make_async_copy(stages[s],
                                     out_hbm.at[pl.ds(wid * BATCH_W + b, 1)],
                                     wsems[s])

    def merge(b, s):
        pv = pairs[s]
        sv = stages[s]
        iota = lax.iota(jnp.int32, LANES)

        def row_body(r, carry):
            rsplat = jnp.full((LANES,), b * SEQ_PAD + r, jnp.int32)
            idv = plsc.load_gather(nidx_v, [rsplat])
            odd = (idv & 1) != 0
            cidv = plsc.load_gather(cidx_v, [rsplat])
            for k in range(NEWS_DIM // LANES):
                lo = pv[r, pl.ds(k * LANES, LANES)]
                hi = pv[r, pl.ds(NEWS_DIM + k * LANES, LANES)]
                sv[0, r, pl.ds(k * LANES, LANES)] = jnp.where(odd, hi, lo)
            cvals = plsc.load_gather(cat_v, [cidv * CATEGORY_DIM + iota])
            sv[0, r, pl.ds(NEWS_DIM, LANES)] = cvals
            return carry

        lax.fori_loop(0, SEQ_LEN, row_body, 0)

    for p in range(LEAD):
        gather_copy(p, p).start()

    def pair_body(g, carry):
        for s in range(NBUF):
            b = g * NBUF + s
            nb = b + LEAD
            @pl.when(nb < BATCH_W)
            def _():
                gather_copy(nb, (s + LEAD) % NBUF).start()
            gather_copy(b, s).wait()
            @pl.when(b >= NBUF)
            def _():
                write_copy(b - NBUF, s).wait()
            merge(b, s)
            write_copy(b, s).start()
        return carry

    lax.fori_loop(0, BATCH_W // NBUF, pair_body, 0)
    for p in range(NBUF):
        b = BATCH_W - NBUF + p
        write_copy(b, b % NBUF).wait()


@jax.jit
def _joint_embed(news_idx2, news_idx, cat_idx, news128, cat_flat):
    mesh = plsc.VectorSubcoreMesh(core_axis_name="c", subcore_axis_name="s")
    f = functools.partial(
        pl.kernel,
        mesh=mesh,
        out_type=jax.ShapeDtypeStruct((BATCH, SEQ_LEN, JOINT_DIM),
                                      jnp.float32),
        scratch_types=[
            pltpu.VMEM((BATCH_W * SEQ_PAD,), jnp.int32),
            pltpu.VMEM((BATCH_W * SEQ_PAD,), jnp.int32),
            pltpu.VMEM((BATCH_W * SEQ_PAD,), jnp.int32),
            pltpu.VMEM((NUM_CATEGORIES * CATEGORY_DIM,), jnp.float32),
            pltpu.VMEM((SEQ_LEN, ROW_PAD), jnp.float32),
            pltpu.VMEM((SEQ_LEN, ROW_PAD), jnp.float32),
            pltpu.VMEM((SEQ_LEN, ROW_PAD), jnp.float32),
            pltpu.VMEM((SEQ_LEN, ROW_PAD), jnp.float32),
            pltpu.VMEM((1, SEQ_LEN, JOINT_DIM), jnp.float32),
            pltpu.VMEM((1, SEQ_LEN, JOINT_DIM), jnp.float32),
            pltpu.VMEM((1, SEQ_LEN, JOINT_DIM), jnp.float32),
            pltpu.VMEM((1, SEQ_LEN, JOINT_DIM), jnp.float32),
            pltpu.SemaphoreType.DMA,
            pltpu.SemaphoreType.DMA,
            pltpu.SemaphoreType.DMA,
            pltpu.SemaphoreType.DMA,
            pltpu.SemaphoreType.DMA,
            pltpu.SemaphoreType.DMA,
            pltpu.SemaphoreType.DMA,
            pltpu.SemaphoreType.DMA,
        ],
        compiler_params=pltpu.CompilerParams(needs_layout_passes=False),
    )(_sc_body)
    return f(news_idx2, news_idx, cat_idx, news128, cat_flat)


def kernel(news_ids, category_ids, news_table, category_table):
    pad = ((0, 0), (0, SEQ_PAD - SEQ_LEN))
    news_idx = jnp.pad(news_ids, pad).reshape(BATCH * SEQ_PAD)
    news_idx2 = news_idx >> 1
    cat_idx = jnp.pad(category_ids, pad).reshape(BATCH * SEQ_PAD)
    news128 = news_table.reshape(NUM_NEWS // 2, ROW_PAD)
    cat_flat = category_table.reshape(NUM_CATEGORIES * CATEGORY_DIM)
    return _joint_embed(news_idx2, news_idx, cat_idx, news128, cat_flat)
